# baseline (speedup 1.0000x reference)
import functools

import jax
import jax.numpy as jnp
from jax import lax
from jax.experimental import pallas as pl
from jax.experimental.pallas import tpu as pltpu
from jax.experimental.pallas import tpu_sc as plsc

NC = 2
NS = 16
NW = NC * NS
B0 = 4096
B1 = 50
D = 32
BW = B0 // NW
SG = 5
NSG = B1 // SG
NBG = BW // 16
NCH = NSG * NBG
CHUNK = 16 * SG


def _emb_kernel(idxT_hbm, tab_hbm, out_hbm,
                idxT_v, idxf_v, subT_v, gbuf_v, obuf_v, gsem, wsem):
    wid = lax.axis_index("s") * NC + lax.axis_index("c")
    bbase = wid * BW

    pltpu.sync_copy(idxT_hbm.at[pl.ds(0, B1), pl.ds(bbase, BW)],
                    idxT_v.at[pl.ds(0, B1)])

    iota = lax.iota(jnp.int32, 16)

    def pidx(c, _):
        g = lax.shift_right_logical(c, 3)
        bg = jnp.bitwise_and(c, 7)
        for sl in range(SG):
            s = g * SG + sl
            svec = jnp.full((16,), s, jnp.int32)
            bcol = bg * 16 + iota
            v = plsc.load_gather(idxT_v, [svec, bcol])
            base = c * CHUNK + sl * 16
            idxf_v[pl.ds(base, 16)] = lax.shift_right_logical(v, 2)
            subT_v[pl.ds(base, 16)] = lax.shift_left(
                jnp.bitwise_and(v, 3), 5)
        return ()

    lax.fori_loop(0, NCH, pidx, (), unroll=False)

    def start_g(c, bs):
        pltpu.async_copy(
            tab_hbm.at[idxf_v.at[pl.ds(c * CHUNK, CHUNK)]],
            gbuf_v.at[bs], gsem)

    def wait_g(c, bs):
        pltpu.make_async_copy(
            tab_hbm.at[idxf_v.at[pl.ds(c * CHUNK, CHUNK)]],
            gbuf_v.at[bs], gsem).wait()

    def out_dst(g):
        return out_hbm.at[pl.ds(g * SG, SG), pl.ds(0, D),
                          pl.ds(bbase, BW)]

    def extract(c, bs, gslot, bg):
        for sl in range(SG):
            sub = subT_v[pl.ds(c * CHUNK + sl * 16, 16)]
            rowv = sl * 16 + iota
            slotv = jnp.full((16,), bs, jnp.int32)
            gv = jnp.full((16,), gslot, jnp.int32)
            slv = jnp.full((16,), sl, jnp.int32)
            bv = bg * 16 + iota

            def jbody(j, carry):
                colv, jv = carry
                v = plsc.load_gather(gbuf_v, [slotv, rowv, colv])
                plsc.store_scatter(obuf_v, [gv, slv, jv, bv], v)
                return (colv + 1, jv + 1)

            lax.fori_loop(0, D, jbody,
                          (sub, jnp.zeros((16,), jnp.int32)), unroll=8)

    def start_w(g, gslot):
        pltpu.async_copy(obuf_v.at[gslot], out_dst(g), wsem)

    def wait_w(g, gslot):
        pltpu.make_async_copy(obuf_v.at[gslot], out_dst(g), wsem).wait()

    start_g(0, 0)

    def gpair(p, _):
        for gsub in range(2):
            g = 2 * p + gsub

            @pl.when(g >= 2)
            def _():
                wait_w(g - 2, gsub)

            for bg in range(NBG):
                c = g * NBG + bg

                @pl.when(c + 1 < NCH)
                def _():
                    start_g(c + 1, (bg + 1) & 1)

                wait_g(c, bg & 1)
                extract(c, bg & 1, gsub, bg)

            start_w(g, gsub)
        return ()

    lax.fori_loop(0, NSG // 2, gpair, (), unroll=False)
    wait_w(NSG - 2, 0)
    wait_w(NSG - 1, 1)


NTC = 7813
TPW = 244


def _tp_kernel(tabT_hbm, tl_hbm, pk_hbm, tin_v, lout_v, rsem, wsem):
    wid = lax.axis_index("s") * NC + lax.axis_index("c")
    iota = lax.iota(jnp.int32, 16)
    jv = [iota, iota + 16]

    def rd(t, bs):
        tc = wid + NW * t
        pltpu.async_copy(
            tabT_hbm.at[pl.ds(0, 32), pl.ds(tc * 128, 128)],
            tin_v.at[bs], rsem)

    def rd_wait(t, bs):
        tc = wid + NW * t
        pltpu.make_async_copy(
            tabT_hbm.at[pl.ds(0, 32), pl.ds(tc * 128, 128)],
            tin_v.at[bs], rsem).wait()

    def transpose_block(bs, nl):
        sp = jnp.full((16,), bs, jnp.int32)
        for L in range(nl):
            for h in range(8):
                col = jnp.full((16,), 4 * L + (h >> 1), jnp.int32)
                v = plsc.load_gather(tin_v, [sp, jv[h & 1], col])
                lout_v[bs, L, pl.ds(h * 16, 16)] = v

    def wr(t, bs):
        tc = wid + NW * t
        pltpu.async_copy(lout_v.at[bs],
                         pk_hbm.at[pl.ds(tc * 32, 32)], wsem)

    def wr_wait(t, bs):
        tc = wid + NW * t
        pltpu.make_async_copy(lout_v.at[bs],
                              pk_hbm.at[pl.ds(tc * 32, 32)], wsem).wait()

    rd(0, 0)

    def tbody(tp, _):
        for par in range(2):
            t = 2 * tp + par

            @pl.when(t + 1 < TPW)
            def _():
                rd(t + 1, 1 - par)

            rd_wait(t, par)

            @pl.when(t >= 2)
            def _():
                wr_wait(t - 2, par)

            transpose_block(par, 32)
            wr(t, par)
        return ()

    lax.fori_loop(0, TPW // 2, tbody, (), unroll=False)
    wr_wait(TPW - 2, 0)
    wr_wait(TPW - 1, 1)

    @pl.when(wid <= 3)
    def _():
        rd(TPW, 0)
        rd_wait(TPW, 0)
        transpose_block(0, 32)
        wr(TPW, 0)
        wr_wait(TPW, 0)

    @pl.when(wid == 4)
    def _():
        pltpu.sync_copy(tl_hbm, tin_v.at[0, pl.ds(0, 16)])
        pltpu.sync_copy(tin_v.at[0, pl.ds(0, 16)],
                        pk_hbm.at[pl.ds(249984, 16)])


def kernel(indices, table):
    idxT = indices.T.astype(jnp.int32)
    tabT = table.T

    mesh = plsc.VectorSubcoreMesh(core_axis_name="c", subcore_axis_name="s")
    k1 = functools.partial(
        pl.kernel,
        mesh=mesh,
        out_type=jax.ShapeDtypeStruct((250000, 128), jnp.float32),
        scratch_types=[
            pltpu.VMEM((2, 32, 128), jnp.float32),
            pltpu.VMEM((2, 32, 128), jnp.float32),
            pltpu.SemaphoreType.DMA,
            pltpu.SemaphoreType.DMA,
        ],
        compiler_params=pltpu.CompilerParams(use_tc_tiling_on_sc=True,
                                             needs_layout_passes=False),
    )(_tp_kernel)
    tail = table[999936:].reshape(16, 128)
    tab = k1(tabT, tail)
    k = functools.partial(
        pl.kernel,
        mesh=mesh,
        out_type=jax.ShapeDtypeStruct((B1, D, B0), jnp.float32),
        scratch_types=[
            pltpu.VMEM((56, BW), jnp.int32),
            pltpu.VMEM((BW * B1,), jnp.int32),
            pltpu.VMEM((BW * B1,), jnp.int32),
            pltpu.VMEM((2, CHUNK, 128), jnp.float32),
            pltpu.VMEM((2, SG, D, BW), jnp.float32),
            pltpu.SemaphoreType.DMA,
            pltpu.SemaphoreType.DMA,
        ],
        compiler_params=pltpu.CompilerParams(use_tc_tiling_on_sc=True,
                                             needs_layout_passes=False),
    )(_emb_kernel)
    out = k(idxT, tab)
    return jnp.transpose(out, (2, 0, 1))


# speedup vs baseline: 1.4733x; 1.4733x over previous
import functools

import jax
import jax.numpy as jnp
from jax import lax
from jax.experimental import pallas as pl
from jax.experimental.pallas import tpu as pltpu
from jax.experimental.pallas import tpu_sc as plsc

NC = 2
NS = 16
NW = NC * NS
B0 = 4096
B1 = 50
D = 32
BW = B0 // NW
SG = 5
NSG = B1 // SG
NBG = BW // 16
NCH = NSG * NBG
CHUNK = 16 * SG


def _emb_kernel(idxT_hbm, tab_hbm, out_hbm,
                idxT_v, idxf_v, subT_v, gbuf_v, obuf_v, gsem, wsem):
    wid = lax.axis_index("s") * NC + lax.axis_index("c")
    bbase = wid * BW

    pltpu.sync_copy(idxT_hbm.at[pl.ds(0, B1), pl.ds(bbase, BW)],
                    idxT_v.at[pl.ds(0, B1)])

    iota = lax.iota(jnp.int32, 16)

    def pidx(c, _):
        g = lax.shift_right_logical(c, 3)
        bg = jnp.bitwise_and(c, 7)
        for sl in range(SG):
            s = g * SG + sl
            svec = jnp.full((16,), s, jnp.int32)
            bcol = bg * 16 + iota
            v = plsc.load_gather(idxT_v, [svec, bcol])
            base = c * CHUNK + sl * 16
            idxf_v[pl.ds(base, 16)] = lax.shift_right_logical(v, 2)
            subT_v[pl.ds(base, 16)] = lax.shift_left(
                jnp.bitwise_and(v, 3), 5)
        return ()

    lax.fori_loop(0, NCH, pidx, (), unroll=False)

    def start_g(c, bs):
        pltpu.async_copy(
            tab_hbm.at[idxf_v.at[pl.ds(c * CHUNK, CHUNK)]],
            gbuf_v.at[bs], gsem)

    def wait_g(c, bs):
        pltpu.make_async_copy(
            tab_hbm.at[idxf_v.at[pl.ds(c * CHUNK, CHUNK)]],
            gbuf_v.at[bs], gsem).wait()

    def out_dst(g):
        return out_hbm.at[pl.ds(g * SG, SG), pl.ds(0, D),
                          pl.ds(bbase, BW)]

    def extract(c, bs, gslot, bg):
        for sl in range(SG):
            sub = subT_v[pl.ds(c * CHUNK + sl * 16, 16)]
            rowv = sl * 16 + iota
            slotv = jnp.full((16,), bs, jnp.int32)
            gv = jnp.full((16,), gslot, jnp.int32)
            slv = jnp.full((16,), sl, jnp.int32)
            bv = bg * 16 + iota

            def jbody(j, carry):
                colv, jv = carry
                v = plsc.load_gather(gbuf_v, [slotv, rowv, colv])
                plsc.store_scatter(obuf_v, [gv, slv, jv, bv], v)
                return (colv + 1, jv + 1)

            lax.fori_loop(0, D, jbody,
                          (sub, jnp.zeros((16,), jnp.int32)), unroll=8)

    def start_w(g, gslot):
        pltpu.async_copy(obuf_v.at[gslot], out_dst(g), wsem)

    def wait_w(g, gslot):
        pltpu.make_async_copy(obuf_v.at[gslot], out_dst(g), wsem).wait()

    start_g(0, 0)

    def gpair(p, _):
        for gsub in range(2):
            g = 2 * p + gsub

            @pl.when(g >= 2)
            def _():
                wait_w(g - 2, gsub)

            for bg in range(NBG):
                c = g * NBG + bg

                @pl.when(c + 1 < NCH)
                def _():
                    start_g(c + 1, (bg + 1) & 1)

                wait_g(c, bg & 1)
                extract(c, bg & 1, gsub, bg)

            start_w(g, gsub)
        return ()

    lax.fori_loop(0, NSG // 2, gpair, (), unroll=False)
    wait_w(NSG - 2, 0)
    wait_w(NSG - 1, 1)


NTC = 7813
TPW = 244


def _tp_kernel(tabT_hbm, tl_hbm, pk_hbm, tin_v, lout_v, rsem, wsem):
    wid = lax.axis_index("s") * NC + lax.axis_index("c")
    iota = lax.iota(jnp.int32, 16)
    jv = [iota, iota + 16]

    def rd(t, bs):
        tc = wid + NW * t
        pltpu.async_copy(
            tabT_hbm.at[pl.ds(0, 32), pl.ds(tc * 128, 128)],
            tin_v.at[bs], rsem)

    def rd_wait(t, bs):
        tc = wid + NW * t
        pltpu.make_async_copy(
            tabT_hbm.at[pl.ds(0, 32), pl.ds(tc * 128, 128)],
            tin_v.at[bs], rsem).wait()

    def transpose_block(bs, nl):
        sp = jnp.full((16,), bs, jnp.int32)

        def blk(k, _):
            L = lax.shift_right_logical(k, 3)
            h = jnp.bitwise_and(k, 7)
            hlow = jnp.bitwise_and(h, 1)
            col = jnp.full(
                (16,), 4 * L + lax.shift_right_logical(h, 1), jnp.int32)
            jvv = iota + hlow * 16
            v = plsc.load_gather(tin_v, [sp, jvv, col])
            plsc.store_scatter(
                lout_v,
                [sp, jnp.full((16,), L, jnp.int32), h * 16 + iota], v)
            return ()

        plsc.parallel_loop(0, nl * 8, 1, unroll=8, carry=())(blk)

    def wr(t, bs):
        tc = wid + NW * t
        pltpu.async_copy(lout_v.at[bs],
                         pk_hbm.at[pl.ds(tc * 32, 32)], wsem)

    def wr_wait(t, bs):
        tc = wid + NW * t
        pltpu.make_async_copy(lout_v.at[bs],
                              pk_hbm.at[pl.ds(tc * 32, 32)], wsem).wait()

    rd(0, 0)

    def tbody(tp, _):
        for par in range(2):
            t = 2 * tp + par

            @pl.when(t + 1 < TPW)
            def _():
                rd(t + 1, 1 - par)

            rd_wait(t, par)

            @pl.when(t >= 2)
            def _():
                wr_wait(t - 2, par)

            transpose_block(par, 32)
            wr(t, par)
        return ()

    lax.fori_loop(0, TPW // 2, tbody, (), unroll=False)
    wr_wait(TPW - 2, 0)
    wr_wait(TPW - 1, 1)

    @pl.when(wid <= 3)
    def _():
        rd(TPW, 0)
        rd_wait(TPW, 0)
        transpose_block(0, 32)
        wr(TPW, 0)
        wr_wait(TPW, 0)

    @pl.when(wid == 4)
    def _():
        pltpu.sync_copy(tl_hbm, tin_v.at[0, pl.ds(0, 16)])
        pltpu.sync_copy(tin_v.at[0, pl.ds(0, 16)],
                        pk_hbm.at[pl.ds(249984, 16)])


def kernel(indices, table):
    idxT = indices.T.astype(jnp.int32)
    tabT = table.T

    mesh = plsc.VectorSubcoreMesh(core_axis_name="c", subcore_axis_name="s")
    k1 = functools.partial(
        pl.kernel,
        mesh=mesh,
        out_type=jax.ShapeDtypeStruct((250000, 128), jnp.float32),
        scratch_types=[
            pltpu.VMEM((2, 32, 128), jnp.float32),
            pltpu.VMEM((2, 32, 128), jnp.float32),
            pltpu.SemaphoreType.DMA,
            pltpu.SemaphoreType.DMA,
        ],
        compiler_params=pltpu.CompilerParams(use_tc_tiling_on_sc=True,
                                             needs_layout_passes=False),
    )(_tp_kernel)
    tail = table[999936:].reshape(16, 128)
    tab = k1(tabT, tail)
    k = functools.partial(
        pl.kernel,
        mesh=mesh,
        out_type=jax.ShapeDtypeStruct((B1, D, B0), jnp.float32),
        scratch_types=[
            pltpu.VMEM((56, BW), jnp.int32),
            pltpu.VMEM((BW * B1,), jnp.int32),
            pltpu.VMEM((BW * B1,), jnp.int32),
            pltpu.VMEM((2, CHUNK, 128), jnp.float32),
            pltpu.VMEM((2, SG, D, BW), jnp.float32),
            pltpu.SemaphoreType.DMA,
            pltpu.SemaphoreType.DMA,
        ],
        compiler_params=pltpu.CompilerParams(use_tc_tiling_on_sc=True,
                                             needs_layout_passes=False),
    )(_emb_kernel)
    out = k(idxT, tab)
    return jnp.transpose(out, (2, 0, 1))


# speedup vs baseline: 1.6679x; 1.1320x over previous
import functools

import jax
import jax.numpy as jnp
from jax import lax
from jax.experimental import pallas as pl
from jax.experimental.pallas import tpu as pltpu
from jax.experimental.pallas import tpu_sc as plsc

NC = 2
NS = 16
NW = NC * NS
B0 = 4096
B1 = 50
D = 32
BW = B0 // NW
SG = 5
NSG = B1 // SG
NBG = BW // 16
NCH = NSG * NBG
CHUNK = 16 * SG


def _emb_kernel(idxT_hbm, tab_hbm, out_hbm,
                idxT_v, idxf_v, subT_v, gbuf_v, obuf_v, gsem, wsem):
    wid = lax.axis_index("s") * NC + lax.axis_index("c")
    bbase = wid * BW

    pltpu.sync_copy(idxT_hbm.at[pl.ds(0, B1), pl.ds(bbase, BW)],
                    idxT_v.at[pl.ds(0, B1)])

    iota = lax.iota(jnp.int32, 16)

    def pidx(c, _):
        g = lax.shift_right_logical(c, 3)
        bg = jnp.bitwise_and(c, 7)
        for sl in range(SG):
            s = g * SG + sl
            svec = jnp.full((16,), s, jnp.int32)
            bcol = bg * 16 + iota
            v = plsc.load_gather(idxT_v, [svec, bcol])
            base = c * CHUNK + sl * 16
            idxf_v[pl.ds(base, 16)] = lax.shift_right_logical(v, 2)
            subT_v[pl.ds(base, 16)] = lax.shift_left(
                jnp.bitwise_and(v, 3), 5)
        return ()

    lax.fori_loop(0, NCH, pidx, (), unroll=False)

    def start_g(c, bs):
        pltpu.async_copy(
            tab_hbm.at[idxf_v.at[pl.ds(c * CHUNK, CHUNK)]],
            gbuf_v.at[bs], gsem)

    def wait_g(c, bs):
        pltpu.make_async_copy(
            tab_hbm.at[idxf_v.at[pl.ds(c * CHUNK, CHUNK)]],
            gbuf_v.at[bs], gsem).wait()

    def out_dst(g):
        return out_hbm.at[pl.ds(g * SG, SG), pl.ds(0, D),
                          pl.ds(bbase, BW)]

    def extract(c, bs, gslot, bg):
        for sl in range(SG):
            sub = subT_v[pl.ds(c * CHUNK + sl * 16, 16)]
            rowv = sl * 16 + iota
            slotv = jnp.full((16,), bs, jnp.int32)
            gv = jnp.full((16,), gslot, jnp.int32)
            slv = jnp.full((16,), sl, jnp.int32)
            bv = bg * 16 + iota

            def jbody(j, carry):
                colv, jv = carry
                v = plsc.load_gather(gbuf_v, [slotv, rowv, colv])
                plsc.store_scatter(obuf_v, [gv, slv, jv, bv], v)
                return (colv + 1, jv + 1)

            lax.fori_loop(0, D, jbody,
                          (sub, jnp.zeros((16,), jnp.int32)), unroll=8)

    def start_w(g, gslot):
        pltpu.async_copy(obuf_v.at[gslot], out_dst(g), wsem)

    def wait_w(g, gslot):
        pltpu.make_async_copy(obuf_v.at[gslot], out_dst(g), wsem).wait()

    start_g(0, 0)

    def gpair(p, _):
        for gsub in range(2):
            g = 2 * p + gsub

            @pl.when(g >= 2)
            def _():
                wait_w(g - 2, gsub)

            for bg in range(NBG):
                c = g * NBG + bg

                @pl.when(c + 1 < NCH)
                def _():
                    start_g(c + 1, (bg + 1) & 1)

                wait_g(c, bg & 1)
                extract(c, bg & 1, gsub, bg)

            start_w(g, gsub)
        return ()

    lax.fori_loop(0, NSG // 2, gpair, (), unroll=False)
    wait_w(NSG - 2, 0)
    wait_w(NSG - 1, 1)


NTC = 7813
TPW = 244


def _tp_kernel(tabT_hbm, tl_hbm, pk_hbm, tin_v, lout_v, rsem, wsem):
    wid = lax.axis_index("s") * NC + lax.axis_index("c")
    iota = lax.iota(jnp.int32, 16)
    jv = [iota, iota + 16]

    def rd(t, bs):
        tc = wid + NW * t
        pltpu.async_copy(
            tabT_hbm.at[pl.ds(0, 32), pl.ds(tc * 128, 128)],
            tin_v.at[bs], rsem)

    def rd_wait(t, bs):
        tc = wid + NW * t
        pltpu.make_async_copy(
            tabT_hbm.at[pl.ds(0, 32), pl.ds(tc * 128, 128)],
            tin_v.at[bs], rsem).wait()

    def transpose_block(bs, nl):
        sp = jnp.full((16,), bs, jnp.int32)

        def blk(L, _):
            base = jnp.full((16,), 4 * L, jnp.int32)
            for h in range(8):
                colv = base + (h >> 1)
                v = plsc.load_gather(tin_v, [sp, jv[h & 1], colv])
                lout_v[bs, L, pl.ds(h * 16, 16)] = v
            return ()

        plsc.parallel_loop(0, nl, 1, unroll=4, carry=())(blk)

    def wr(t, bs):
        tc = wid + NW * t
        pltpu.async_copy(lout_v.at[bs],
                         pk_hbm.at[pl.ds(tc * 32, 32)], wsem)

    def wr_wait(t, bs):
        tc = wid + NW * t
        pltpu.make_async_copy(lout_v.at[bs],
                              pk_hbm.at[pl.ds(tc * 32, 32)], wsem).wait()

    rd(0, 0)

    def tbody(tp, _):
        for par in range(2):
            t = 2 * tp + par

            @pl.when(t + 1 < TPW)
            def _():
                rd(t + 1, 1 - par)

            rd_wait(t, par)

            @pl.when(t >= 2)
            def _():
                wr_wait(t - 2, par)

            transpose_block(par, 32)
            wr(t, par)
        return ()

    lax.fori_loop(0, TPW // 2, tbody, (), unroll=False)
    wr_wait(TPW - 2, 0)
    wr_wait(TPW - 1, 1)

    @pl.when(wid <= 3)
    def _():
        rd(TPW, 0)
        rd_wait(TPW, 0)
        transpose_block(0, 32)
        wr(TPW, 0)
        wr_wait(TPW, 0)

    @pl.when(wid == 4)
    def _():
        pltpu.sync_copy(tl_hbm, tin_v.at[0, pl.ds(0, 16)])
        pltpu.sync_copy(tin_v.at[0, pl.ds(0, 16)],
                        pk_hbm.at[pl.ds(249984, 16)])


def kernel(indices, table):
    idxT = indices.T.astype(jnp.int32)
    tabT = table.T

    mesh = plsc.VectorSubcoreMesh(core_axis_name="c", subcore_axis_name="s")
    k1 = functools.partial(
        pl.kernel,
        mesh=mesh,
        out_type=jax.ShapeDtypeStruct((250000, 128), jnp.float32),
        scratch_types=[
            pltpu.VMEM((2, 32, 128), jnp.float32),
            pltpu.VMEM((2, 32, 128), jnp.float32),
            pltpu.SemaphoreType.DMA,
            pltpu.SemaphoreType.DMA,
        ],
        compiler_params=pltpu.CompilerParams(use_tc_tiling_on_sc=True,
                                             needs_layout_passes=False),
    )(_tp_kernel)
    tail = table[999936:].reshape(16, 128)
    tab = k1(tabT, tail)
    k = functools.partial(
        pl.kernel,
        mesh=mesh,
        out_type=jax.ShapeDtypeStruct((B1, D, B0), jnp.float32),
        scratch_types=[
            pltpu.VMEM((56, BW), jnp.int32),
            pltpu.VMEM((BW * B1,), jnp.int32),
            pltpu.VMEM((BW * B1,), jnp.int32),
            pltpu.VMEM((2, CHUNK, 128), jnp.float32),
            pltpu.VMEM((2, SG, D, BW), jnp.float32),
            pltpu.SemaphoreType.DMA,
            pltpu.SemaphoreType.DMA,
        ],
        compiler_params=pltpu.CompilerParams(use_tc_tiling_on_sc=True,
                                             needs_layout_passes=False),
    )(_emb_kernel)
    out = k(idxT, tab)
    return jnp.transpose(out, (2, 0, 1))
